# baseline (speedup 1.0000x reference)
import functools

import jax
import jax.numpy as jnp
from jax import lax
from jax.experimental import pallas as pl
from jax.experimental.pallas import tpu as pltpu


def _make_kernel(n_stage, th, bm, rc):
    n_chunk = bm // rc

    def _ffn_kernel(x_ref, w1_ref, b1_ref, w2_ref, b2_ref, o_ref,
                    w1b_ref, w2b_ref):
        p = pl.program_id(0)

        @pl.when(p < n_stage)
        def _stage():
            w1b_ref[:, pl.ds(p * th, th)] = w1_ref[...].astype(jnp.bfloat16)
            w2b_ref[pl.ds(p * th, th), :] = w2_ref[...].astype(jnp.bfloat16)

        @pl.when(p >= n_stage)
        def _bulk():
            for r in range(n_chunk):
                xb = x_ref[r * rc:(r + 1) * rc, :].astype(jnp.bfloat16)
                t = jnp.dot(xb, w1b_ref[...], preferred_element_type=jnp.float32)
                t = t + b1_ref[...]
                t = 0.5 * t * (1.0 + lax.erf(t * 0.7071067811865476))
                o_ref[r * rc:(r + 1) * rc, :] = jnp.dot(
                    t.astype(jnp.bfloat16), w2b_ref[...],
                    preferred_element_type=jnp.float32) + b2_ref[...]

    return _ffn_kernel


@functools.partial(jax.jit, static_argnames=("block_rows", "block_hidden", "row_chunk"))
def kernel(x, w1, b1, w2, b2, *, block_rows=1024, block_hidden=1024, row_chunk=256):
    orig_lead = x.shape[:-1]
    C_in = x.shape[-1]
    H = w1.shape[1]
    C_out = w2.shape[1]
    rows = 1
    for d in orig_lead:
        rows *= d

    x2 = x.reshape(rows, C_in)
    b1r = b1.reshape(1, H)
    b2r = b2.reshape(1, C_out)

    bm = min(block_rows, rows)
    n_row = rows // bm
    th = min(block_hidden, H)
    n_stage = H // th
    n_steps = n_stage + n_row

    ns = n_stage

    out2d = pl.pallas_call(
        _make_kernel(n_stage, th, bm, min(row_chunk, bm)),
        out_shape=jax.ShapeDtypeStruct((rows, C_out), jnp.float32),
        grid=(n_steps,),
        in_specs=[
            pl.BlockSpec((bm, C_in), lambda p: (jnp.maximum(p - ns, 0), 0)),
            pl.BlockSpec((C_in, th), lambda p: (0, jnp.minimum(p, ns - 1))),
            pl.BlockSpec((1, H), lambda p: (0, 0)),
            pl.BlockSpec((th, C_out), lambda p: (jnp.minimum(p, ns - 1), 0)),
            pl.BlockSpec((1, C_out), lambda p: (0, 0)),
        ],
        out_specs=pl.BlockSpec((bm, C_out),
                               lambda p: (jnp.maximum(p - ns, 0), 0)),
        scratch_shapes=[
            pltpu.VMEM((C_in, H), jnp.bfloat16),
            pltpu.VMEM((H, C_out), jnp.bfloat16),
        ],
        compiler_params=pltpu.CompilerParams(
            dimension_semantics=("arbitrary",),
            vmem_limit_bytes=61 << 20,
        ),
    )(x2, w1, b1r, w2, b2r)

    return out2d.reshape(*orig_lead, C_out).astype(x.dtype)


# speedup vs baseline: 1.0865x; 1.0865x over previous
import functools

import jax
import jax.numpy as jnp
from jax import lax
from jax.experimental import pallas as pl
from jax.experimental.pallas import tpu as pltpu


def _gelu(t):
    return 0.5 * t * (1.0 + lax.erf(t * 0.7071067811865476))


def _make_kernel(n_stream, th):
    def _ffn_kernel(x_ref, w1_ref, b1t_ref, b1f_ref, w2_ref, b2_ref, o_ref,
                    w1b_ref, w2b_ref, xb_ref):
        p = pl.program_id(0)

        @pl.when(p == 0)
        def _first():
            xb_ref[...] = x_ref[...].astype(jnp.bfloat16)
            o_ref[...] = jnp.broadcast_to(b2_ref[...], o_ref.shape)

        @pl.when(p < n_stream)
        def _stream():
            w1t = w1_ref[...].astype(jnp.bfloat16)
            w2t = w2_ref[...].astype(jnp.bfloat16)
            w1b_ref[:, pl.ds(p * th, th)] = w1t
            w2b_ref[pl.ds(p * th, th), :] = w2t
            t = jnp.dot(xb_ref[...], w1t, preferred_element_type=jnp.float32)
            t = _gelu(t + b1t_ref[...])
            o_ref[...] += jnp.dot(t.astype(jnp.bfloat16), w2t,
                                  preferred_element_type=jnp.float32)

        @pl.when(p >= n_stream)
        def _bulk():
            xb = x_ref[...].astype(jnp.bfloat16)
            t = jnp.dot(xb, w1b_ref[...], preferred_element_type=jnp.float32)
            t = _gelu(t + b1f_ref[...])
            o_ref[...] = jnp.dot(t.astype(jnp.bfloat16), w2b_ref[...],
                                 preferred_element_type=jnp.float32) + b2_ref[...]

    return _ffn_kernel


@functools.partial(jax.jit, static_argnames=("block_rows", "block_hidden"))
def kernel(x, w1, b1, w2, b2, *, block_rows=512, block_hidden=1024):
    orig_lead = x.shape[:-1]
    C_in = x.shape[-1]
    H = w1.shape[1]
    C_out = w2.shape[1]
    rows = 1
    for d in orig_lead:
        rows *= d

    x2 = x.reshape(rows, C_in)
    b1r = b1.reshape(1, H)
    b2r = b2.reshape(1, C_out)

    bm = min(block_rows, rows)
    n_row = rows // bm
    th = min(block_hidden, H)
    n_stream = H // th
    n_steps = n_stream + n_row - 1

    ns = n_stream

    out2d = pl.pallas_call(
        _make_kernel(n_stream, th),
        out_shape=jax.ShapeDtypeStruct((rows, C_out), jnp.float32),
        grid=(n_steps,),
        in_specs=[
            pl.BlockSpec((bm, C_in),
                         lambda p: (jnp.maximum(p - (ns - 1), 0), 0)),
            pl.BlockSpec((C_in, th), lambda p: (0, jnp.minimum(p, ns - 1))),
            pl.BlockSpec((1, th), lambda p: (0, jnp.minimum(p, ns - 1))),
            pl.BlockSpec((1, H), lambda p: (0, 0)),
            pl.BlockSpec((th, C_out), lambda p: (jnp.minimum(p, ns - 1), 0)),
            pl.BlockSpec((1, C_out), lambda p: (0, 0)),
        ],
        out_specs=pl.BlockSpec((bm, C_out),
                               lambda p: (jnp.maximum(p - (ns - 1), 0), 0)),
        scratch_shapes=[
            pltpu.VMEM((C_in, H), jnp.bfloat16),
            pltpu.VMEM((H, C_out), jnp.bfloat16),
            pltpu.VMEM((bm, C_in), jnp.bfloat16),
        ],
        compiler_params=pltpu.CompilerParams(
            dimension_semantics=("arbitrary",),
            vmem_limit_bytes=61 << 20,
        ),
    )(x2, w1, b1r, b1r, w2, b2r)

    return out2d.reshape(*orig_lead, C_out).astype(x.dtype)
